# baseline (speedup 1.0000x reference)
import functools

import jax
import jax.numpy as jnp
from jax import lax
from jax.experimental import pallas as pl
from jax.experimental.pallas import tpu as pltpu
from jax.experimental.pallas import tpu_sc as plsc

E = 2049
EP = 2064
R = 448
RPW = 16
NACT = R // RPW


def _densify_sc(vals_f, vals_t, rows_f, cols_f, rows_t, cols_t):
    nnz_t = rows_t.shape[0]
    nnz_f = rows_f.shape[0]
    nt = (nnz_t + 15) // 16
    nf = (nnz_f + 15) // 16

    mesh = plsc.VectorSubcoreMesh(core_axis_name="c", subcore_axis_name="s")

    @functools.partial(
        pl.kernel,
        mesh=mesh,
        out_type=[jax.ShapeDtypeStruct((R, EP), jnp.float32)] * 2,
        scratch_types=[
            pltpu.VMEM((nnz_t,), jnp.int32),
            pltpu.VMEM((nnz_t,), jnp.int32),
            pltpu.VMEM((nnz_t,), jnp.float32),
            pltpu.VMEM((nnz_f,), jnp.int32),
            pltpu.VMEM((nnz_f,), jnp.int32),
            pltpu.VMEM((nnz_f,), jnp.float32),
            pltpu.VMEM((RPW, EP), jnp.float32),
            pltpu.VMEM((RPW, EP), jnp.float32),
        ],
        compiler_params=pltpu.CompilerParams(needs_layout_passes=False),
    )
    def build(rt_h, ct_h, vt_h, rf_h, cf_h, vf_h, mt_h, mf_h,
              rt_v, ct_v, vt_v, rf_v, cf_v, vf_v, bt, bf):
        wid = lax.axis_index("s") * 2 + lax.axis_index("c")
        base = wid * RPW

        @pl.when(wid < NACT)
        def _():
            pltpu.sync_copy(rt_h, rt_v)
            pltpu.sync_copy(ct_h, ct_v)
            pltpu.sync_copy(vt_h, vt_v)
            pltpu.sync_copy(rf_h, rf_v)
            pltpu.sync_copy(cf_h, cf_v)
            pltpu.sync_copy(vf_h, vf_v)

            zeros16 = jnp.zeros((16,), jnp.float32)
            for r in range(RPW):
                def zrow(j, c, r=r):
                    bt[r, pl.ds(j * 16, 16)] = zeros16
                    bf[r, pl.ds(j * 16, 16)] = zeros16
                    return c
                lax.fori_loop(0, EP // 16, zrow, 0)

            def scat(k, c, rv, cv, vv, buf, last):
                i = jnp.minimum(k * 16, last)
                r16 = rv[pl.ds(i, 16)]
                c16 = cv[pl.ds(i, 16)]
                v16 = vv[pl.ds(i, 16)]
                lr = r16 - base
                m = (lr >= 0) & (lr < RPW)
                lrc = jnp.clip(lr, 0, RPW - 1)
                plsc.store_scatter(buf, [lrc, c16], v16, mask=m)
                return c

            lax.fori_loop(0, nt, functools.partial(
                scat, rv=rt_v, cv=ct_v, vv=vt_v, buf=bt, last=nnz_t - 16), 0)
            lax.fori_loop(0, nf, functools.partial(
                scat, rv=rf_v, cv=cf_v, vv=vf_v, buf=bf, last=nnz_f - 16), 0)

            pltpu.sync_copy(bt, mt_h.at[pl.ds(base, RPW), :])
            pltpu.sync_copy(bf, mf_h.at[pl.ds(base, RPW), :])

    return build(rows_t, cols_t, vals_t, rows_f, cols_f, vals_f)


def _mm_body(x_ref, s_ref, mt_ref, mf_ref, ot_ref, of_ref):
    mt = mt_ref[...][:, :E].astype(jnp.bfloat16)
    mf = mf_ref[...][:, :E].astype(jnp.bfloat16)
    dn = (((1,), (1,)), ((), ()))
    ot_ref[0] = lax.dot_general(mt, x_ref[0].astype(jnp.bfloat16), dn,
                                preferred_element_type=jnp.float32)
    of_ref[0] = lax.dot_general(mf, s_ref[0].astype(jnp.bfloat16), dn,
                                preferred_element_type=jnp.float32)


def kernel(ceps, spec, vals_f, vals_t, rows_f, cols_f, rows_t, cols_t):
    batch, steps, _ = ceps.shape

    mt, mf = _densify_sc(vals_f, vals_t, rows_f, cols_f, rows_t, cols_t)

    ot, of = pl.pallas_call(
        _mm_body,
        grid=(batch,),
        in_specs=[
            pl.BlockSpec((1, steps, E), lambda i: (i, 0, 0)),
            pl.BlockSpec((1, steps, E), lambda i: (i, 0, 0)),
            pl.BlockSpec((R, EP), lambda i: (0, 0)),
            pl.BlockSpec((R, EP), lambda i: (0, 0)),
        ],
        out_specs=[
            pl.BlockSpec((1, R, steps), lambda i: (i, 0, 0)),
            pl.BlockSpec((1, R, steps), lambda i: (i, 0, 0)),
        ],
        out_shape=[jax.ShapeDtypeStruct((batch, R, steps), jnp.float32)] * 2,
        compiler_params=pltpu.CompilerParams(vmem_limit_bytes=50 * 2**20,
                                             allow_input_fusion=[True, True, False, False]),
    )(ceps, spec, mt, mf)

    return jnp.transpose(ot, (0, 2, 1)), jnp.transpose(of, (0, 2, 1))


# speedup vs baseline: 1.0030x; 1.0030x over previous
import functools

import jax
import jax.numpy as jnp
from jax import lax
from jax.experimental import pallas as pl
from jax.experimental.pallas import tpu as pltpu
from jax.experimental.pallas import tpu_sc as plsc

E = 2049
EP = 2064
R = 448
RPW = 16
NACT = R // RPW


def _densify_sc(vals_f, vals_t, rows_f, cols_f, rows_t, cols_t):
    nnz_t = rows_t.shape[0]
    nnz_f = rows_f.shape[0]
    nt = (nnz_t + 15) // 16
    nf = (nnz_f + 15) // 16

    mesh = plsc.VectorSubcoreMesh(core_axis_name="c", subcore_axis_name="s")

    @functools.partial(
        pl.kernel,
        mesh=mesh,
        out_type=[jax.ShapeDtypeStruct((R, EP), jnp.float32)] * 2,
        scratch_types=[
            pltpu.VMEM((nnz_t,), jnp.int32),
            pltpu.VMEM((nnz_t,), jnp.int32),
            pltpu.VMEM((nnz_t,), jnp.float32),
            pltpu.VMEM((nnz_f,), jnp.int32),
            pltpu.VMEM((nnz_f,), jnp.int32),
            pltpu.VMEM((nnz_f,), jnp.float32),
            pltpu.VMEM((RPW, EP), jnp.float32),
            pltpu.VMEM((RPW, EP), jnp.float32),
        ],
        compiler_params=pltpu.CompilerParams(needs_layout_passes=False),
    )
    def build(rt_h, ct_h, vt_h, rf_h, cf_h, vf_h, mt_h, mf_h,
              rt_v, ct_v, vt_v, rf_v, cf_v, vf_v, bt, bf):
        wid = lax.axis_index("s") * 2 + lax.axis_index("c")
        base = wid * RPW

        @pl.when(wid < NACT)
        def _():
            pltpu.sync_copy(rt_h, rt_v)
            pltpu.sync_copy(ct_h, ct_v)
            pltpu.sync_copy(vt_h, vt_v)
            pltpu.sync_copy(rf_h, rf_v)
            pltpu.sync_copy(cf_h, cf_v)
            pltpu.sync_copy(vf_h, vf_v)

            zeros16 = jnp.zeros((16,), jnp.float32)
            for r in range(RPW):
                def zrow(j, c, r=r):
                    bt[r, pl.ds(j * 16, 16)] = zeros16
                    bf[r, pl.ds(j * 16, 16)] = zeros16
                    return c
                lax.fori_loop(0, EP // 16, zrow, 0)

            def scat(k, c, rv, cv, vv, buf, last):
                i = jnp.minimum(k * 16, last)
                r16 = rv[pl.ds(i, 16)]
                c16 = cv[pl.ds(i, 16)]
                v16 = vv[pl.ds(i, 16)]
                lr = r16 - base
                m = (lr >= 0) & (lr < RPW)
                lrc = jnp.clip(lr, 0, RPW - 1)
                plsc.store_scatter(buf, [lrc, c16], v16, mask=m)
                return c

            lax.fori_loop(0, nt, functools.partial(
                scat, rv=rt_v, cv=ct_v, vv=vt_v, buf=bt, last=nnz_t - 16), 0)
            lax.fori_loop(0, nf, functools.partial(
                scat, rv=rf_v, cv=cf_v, vv=vf_v, buf=bf, last=nnz_f - 16), 0)

            pltpu.sync_copy(bt, mt_h.at[pl.ds(base, RPW), :])
            pltpu.sync_copy(bf, mf_h.at[pl.ds(base, RPW), :])

    return build(rows_t, cols_t, vals_t, rows_f, cols_f, vals_f)


def _mm_body(x_ref, s_ref, mt_ref, mf_ref, ot_ref, of_ref):
    mt = mt_ref[...][:, :E].astype(jnp.bfloat16)
    mf = mf_ref[...][:, :E].astype(jnp.bfloat16)
    dn = (((1,), (1,)), ((), ()))
    ot_ref[0] = lax.dot_general(mt, x_ref[0].astype(jnp.bfloat16), dn,
                                preferred_element_type=jnp.float32)
    of_ref[0] = lax.dot_general(mf, s_ref[0].astype(jnp.bfloat16), dn,
                                preferred_element_type=jnp.float32)


def kernel(ceps, spec, vals_f, vals_t, rows_f, cols_f, rows_t, cols_t):
    batch, steps, _ = ceps.shape

    mt, mf = _densify_sc(vals_f, vals_t, rows_f, cols_f, rows_t, cols_t)

    ot, of = pl.pallas_call(
        _mm_body,
        grid=(batch,),
        in_specs=[
            pl.BlockSpec((1, steps, E), lambda i: (i, 0, 0)),
            pl.BlockSpec((1, steps, E), lambda i: (i, 0, 0)),
            pl.BlockSpec((R, EP), lambda i: (0, 0)),
            pl.BlockSpec((R, EP), lambda i: (0, 0)),
        ],
        out_specs=[
            pl.BlockSpec((1, R, steps), lambda i: (i, 0, 0)),
            pl.BlockSpec((1, R, steps), lambda i: (i, 0, 0)),
        ],
        out_shape=[jax.ShapeDtypeStruct((batch, R, steps), jnp.float32)] * 2,
        compiler_params=pltpu.CompilerParams(vmem_limit_bytes=50 * 2**20),
    )(ceps, spec, mt, mf)

    return jnp.transpose(ot, (0, 2, 1)), jnp.transpose(of, (0, 2, 1))
